# baseline (speedup 1.0000x reference)
import functools

import jax
import jax.numpy as jnp
from jax import lax
from jax.experimental import pallas as pl
from jax.experimental.pallas import tpu as pltpu
from jax.experimental.pallas import tpu_sc as plsc

ALPHA = 3.0
KTOP = 20
NEG = -3.0e38


def _sc_gather(tab, idx):
    b = idx.shape[0]
    d = tab.shape[1]
    try:
        info = plsc.get_sparse_core_info()
        nc, ns = info.num_cores, info.num_subcores
    except ValueError:
        nc, ns = 2, 16
    nw = nc * ns
    chunk = 104 if b >= 104 * nw else 8
    nch = b // (chunk * nw)
    bpw = chunk * nch
    tail = b - bpw * nw
    mesh = plsc.VectorSubcoreMesh(core_axis_name="c", subcore_axis_name="s",
                                  num_cores=nc, num_subcores=ns)

    @functools.partial(
        pl.kernel, mesh=mesh,
        out_type=jax.ShapeDtypeStruct((b, d), jnp.float32),
        scratch_types=[
            pltpu.VMEM((chunk,), jnp.int32),
            pltpu.VMEM((chunk, d), jnp.float32),
            pltpu.VMEM((max(tail, 8),), jnp.int32),
            pltpu.VMEM((max(tail, 8), d), jnp.float32),
            pltpu.SemaphoreType.DMA,
        ],
    )
    def k(tab_hbm, idx_hbm, o_hbm, idx_v, rows_v, idxt_v, rowst_v, sem):
        wid = lax.axis_index("s") * nc + lax.axis_index("c")
        for ci in range(nch):
            base = wid * bpw + ci * chunk
            pltpu.sync_copy(idx_hbm.at[pl.ds(base, chunk)], idx_v)
            pltpu.async_copy(tab_hbm.at[idx_v], rows_v, sem).wait()
            pltpu.sync_copy(rows_v, o_hbm.at[pl.ds(base, chunk)])
        if tail:
            @pl.when(wid == 0)
            def _tail():
                tb = nw * bpw
                pltpu.sync_copy(idx_hbm.at[pl.ds(tb, tail)], idxt_v)
                pltpu.async_copy(tab_hbm.at[idxt_v], rowst_v, sem).wait()
                pltpu.sync_copy(rowst_v, o_hbm.at[pl.ds(tb, tail)])

    return k(tab, idx)


def _nodevec_body(emb1_ref, emb2_ref, w1_ref, b1_ref, w2_ref, b2_ref,
                  c1_ref, c2_ref):
    dn = (((1,), (1,)), ((), ()))
    y1 = jax.lax.dot_general(emb1_ref[...], w1_ref[...], dn,
                             preferred_element_type=jnp.float32)
    y2 = jax.lax.dot_general(emb2_ref[...], w2_ref[...], dn,
                             preferred_element_type=jnp.float32)
    n1 = jnp.tanh(ALPHA * (y1 + b1_ref[...]))
    n2 = jnp.tanh(ALPHA * (y2 + b2_ref[...]))
    c1_ref[...] = jnp.concatenate([n1, -n2], axis=1)
    c2_ref[...] = jnp.concatenate([n2, n1], axis=1)


def _pick_cw(n):
    best = 1
    d = 1
    while d * d <= n:
        if n % d == 0:
            best = d
        d += 1
    return best


def _adj_body(c1b_ref, c2_ref, epos_ref, echk_ref, tri_ref, tri2_ref,
              jchunk_ref, jcol_ref, out_ref):
    r = c1b_ref.shape[0]
    n = c2_ref.shape[0]
    dn1 = (((1,), (1,)), ((), ()))
    dn0 = (((1,), (0,)), ((), ()))
    a = jax.lax.dot_general(c1b_ref[...], c2_ref[...], dn1,
                            preferred_element_type=jnp.float32)
    adj = jnp.where(a > 0.0, jnp.tanh(ALPHA * a), 0.0)

    kf = jnp.float32(KTOP)
    m1 = jnp.max(adj, axis=1, keepdims=True)
    cnt1 = jnp.sum((adj >= m1).astype(jnp.float32), axis=1, keepdims=True)
    hit1 = cnt1 >= kf
    v20_0 = jnp.where(hit1, m1, 0.0)
    need_0 = jnp.where(hit1, kf, 0.0)

    def cond(st):
        cum = st[0]
        it = st[-1]
        return jnp.logical_and(jnp.any(cum < kf), it < KTOP)

    def body(st):
        cum, thr, v20, need, it = st
        m = jnp.max(jnp.where(adj < thr, adj, NEG), axis=1, keepdims=True)
        cnt = jnp.sum((adj == m).astype(jnp.float32), axis=1, keepdims=True)
        active = cum < kf
        hit = active & (cum + cnt >= kf)
        v20 = jnp.where(hit, m, v20)
        need = jnp.where(hit, kf - cum, need)
        thr = jnp.where(active, m, thr)
        cum = cum + jnp.where(active, cnt, 0.0)
        return cum, thr, v20, need, it + 1

    init = (cnt1, m1, v20_0, need_0, jnp.int32(0))
    _, _, v20, need_f, _ = jax.lax.while_loop(cond, body, init)

    tie = adj == v20
    tie_b = tie.astype(jnp.bfloat16)
    s = jax.lax.dot_general(tie_b, echk_ref[...], dn0,
                            preferred_element_type=jnp.float32)
    p = jax.lax.dot_general(s, tri2_ref[...], dn0,
                            preferred_element_type=jnp.float32)

    ch = s.shape[1]
    cw = epos_ref.shape[1]
    cstar = jnp.sum((p < need_f).astype(jnp.int32), axis=1,
                    keepdims=True) - 1
    onehot = (jax.lax.broadcasted_iota(jnp.int32, (r, ch), 1) == cstar)
    quota = need_f - jnp.sum(jnp.where(onehot, p, 0.0), axis=1, keepdims=True)

    extb = jnp.where(jchunk_ref[...] == cstar, tie_b, jnp.bfloat16(0.0))
    ext = jax.lax.dot_general(extb, epos_ref[...], dn0,
                              preferred_element_type=jnp.float32)
    rank_in = jax.lax.dot_general(ext, tri_ref[...], dn0,
                                  preferred_element_type=jnp.float32)
    hitp = (ext > 0.5) & (rank_in == quota - 1.0)
    pidx = jax.lax.broadcasted_iota(jnp.int32, (r, cw), 1)
    pstar = jnp.sum(jnp.where(hitp, pidx, 0), axis=1, keepdims=True)
    jstar = cstar * cw + pstar

    sel = (adj > v20) | (tie & (jcol_ref[...] <= jstar))
    out_ref[...] = jnp.where(sel, adj, 0.0)


def kernel(idx, emb1, emb2, W1, b1, W2, b2):
    n = idx.shape[0]
    d = emb1.shape[1]
    tab = jnp.concatenate(
        [emb1, emb2, jnp.zeros((n, 128 - 2 * d), jnp.float32)], axis=1)
    gathered = _sc_gather(tab, idx.astype(jnp.int32))
    e1 = gathered[:, :d]
    e2 = gathered[:, d:2 * d]
    b1r = b1.reshape(1, d)
    b2r = b2.reshape(1, d)

    c1, c2 = pl.pallas_call(
        _nodevec_body,
        out_shape=(jax.ShapeDtypeStruct((n, 2 * d), jnp.float32),
                   jax.ShapeDtypeStruct((n, 2 * d), jnp.float32)),
    )(e1, e2, W1, b1r, W2, b2r)

    cw = _pick_cw(n)
    ch = n // cw
    jn = jnp.arange(n, dtype=jnp.int32)
    epos = (jn[:, None] % cw == jnp.arange(cw)[None, :]).astype(jnp.bfloat16)
    echk = (jn[:, None] // cw == jnp.arange(ch)[None, :]).astype(jnp.bfloat16)
    tri = (jnp.arange(cw)[:, None] < jnp.arange(cw)[None, :]).astype(jnp.float32)
    tri2 = (jnp.arange(ch)[:, None] < jnp.arange(ch)[None, :]).astype(jnp.float32)
    jchunk = (jn // cw).astype(jnp.int32).reshape(1, n)
    jcol = jn.reshape(1, n)

    rows = 400 if n % 400 == 0 else 8
    grid = n // rows
    adj = pl.pallas_call(
        _adj_body,
        compiler_params=pltpu.CompilerParams(
            vmem_limit_bytes=100 * 1024 * 1024),
        grid=(grid,),
        in_specs=[
            pl.BlockSpec((rows, 2 * d), lambda i: (i, 0)),
            pl.BlockSpec((n, 2 * d), lambda i: (0, 0)),
            pl.BlockSpec((n, cw), lambda i: (0, 0)),
            pl.BlockSpec((n, ch), lambda i: (0, 0)),
            pl.BlockSpec((cw, cw), lambda i: (0, 0)),
            pl.BlockSpec((ch, ch), lambda i: (0, 0)),
            pl.BlockSpec((1, n), lambda i: (0, 0)),
            pl.BlockSpec((1, n), lambda i: (0, 0)),
        ],
        out_specs=pl.BlockSpec((rows, n), lambda i: (i, 0)),
        out_shape=jax.ShapeDtypeStruct((n, n), jnp.float32),
    )(c1, c2, epos, echk, tri, tri2, jchunk, jcol)
    return adj


# speedup vs baseline: 1.1220x; 1.1220x over previous
import functools

import jax
import jax.numpy as jnp
from jax import lax
from jax.experimental import pallas as pl
from jax.experimental.pallas import tpu as pltpu
from jax.experimental.pallas import tpu_sc as plsc

ALPHA = 3.0
KTOP = 20
NEG = -3.0e38


def _sc_gather(tab, idx):
    b = idx.shape[0]
    d = tab.shape[1]
    try:
        info = plsc.get_sparse_core_info()
        nc, ns = info.num_cores, info.num_subcores
    except ValueError:
        nc, ns = 2, 16
    nw = nc * ns
    chunk = 104 if b >= 104 * nw else 8
    nch = b // (chunk * nw)
    bpw = chunk * nch
    tail = b - bpw * nw
    mesh = plsc.VectorSubcoreMesh(core_axis_name="c", subcore_axis_name="s",
                                  num_cores=nc, num_subcores=ns)

    @functools.partial(
        pl.kernel, mesh=mesh,
        out_type=jax.ShapeDtypeStruct((b, d), jnp.float32),
        scratch_types=[
            pltpu.VMEM((chunk,), jnp.int32),
            pltpu.VMEM((chunk, d), jnp.float32),
            pltpu.VMEM((max(tail, 8),), jnp.int32),
            pltpu.VMEM((max(tail, 8), d), jnp.float32),
            pltpu.SemaphoreType.DMA,
        ],
    )
    def k(tab_hbm, idx_hbm, o_hbm, idx_v, rows_v, idxt_v, rowst_v, sem):
        wid = lax.axis_index("s") * nc + lax.axis_index("c")
        for ci in range(nch):
            base = wid * bpw + ci * chunk
            pltpu.sync_copy(idx_hbm.at[pl.ds(base, chunk)], idx_v)
            pltpu.async_copy(tab_hbm.at[idx_v], rows_v, sem).wait()
            pltpu.sync_copy(rows_v, o_hbm.at[pl.ds(base, chunk)])
        if tail:
            @pl.when(wid == 0)
            def _tail():
                tb = nw * bpw
                pltpu.sync_copy(idx_hbm.at[pl.ds(tb, tail)], idxt_v)
                pltpu.async_copy(tab_hbm.at[idxt_v], rowst_v, sem).wait()
                pltpu.sync_copy(rowst_v, o_hbm.at[pl.ds(tb, tail)])

    return k(tab, idx)


def _nodevec_body(emb1_ref, emb2_ref, w1_ref, b1_ref, w2_ref, b2_ref,
                  c1_ref, c2_ref):
    dn = (((1,), (1,)), ((), ()))
    y1 = jax.lax.dot_general(emb1_ref[...], w1_ref[...], dn,
                             preferred_element_type=jnp.float32)
    y2 = jax.lax.dot_general(emb2_ref[...], w2_ref[...], dn,
                             preferred_element_type=jnp.float32)
    n1 = jnp.tanh(ALPHA * (y1 + b1_ref[...]))
    n2 = jnp.tanh(ALPHA * (y2 + b2_ref[...]))
    c1_ref[...] = jnp.concatenate([n1, -n2], axis=1)
    c2_ref[...] = jnp.concatenate([n2, n1], axis=1)


def _pick_cw(n):
    best = 1
    d = 1
    while d * d <= n:
        if n % d == 0:
            best = d
        d += 1
    return best


def _adj_body(c1b_ref, c2_ref, epos_ref, echk_ref, tri_ref, tri2_ref,
              jchunk_ref, jcol_ref, out_ref):
    r = c1b_ref.shape[0]
    n = c2_ref.shape[0]
    dn1 = (((1,), (1,)), ((), ()))
    dn0 = (((1,), (0,)), ((), ()))
    a = jax.lax.dot_general(c1b_ref[...], c2_ref[...], dn1,
                            preferred_element_type=jnp.float32)
    adj = jnp.where(a > 0.0, jnp.tanh(ALPHA * a), 0.0)

    kf = jnp.float32(KTOP)
    m1 = jnp.max(adj, axis=1, keepdims=True)
    cnt1 = jnp.sum((adj >= m1).astype(jnp.float32), axis=1, keepdims=True)
    hit1 = cnt1 >= kf
    v20_0 = jnp.where(hit1, m1, 0.0)
    need_0 = jnp.where(hit1, kf, 0.0)

    def cond(st):
        cum = st[0]
        it = st[-1]
        return jnp.logical_and(jnp.any(cum < kf), it < KTOP)

    def body(st):
        cum, thr, v20, need, it = st
        m = jnp.max(jnp.where(adj < thr, adj, NEG), axis=1, keepdims=True)
        cnt = jnp.sum((adj == m).astype(jnp.float32), axis=1, keepdims=True)
        active = cum < kf
        hit = active & (cum + cnt >= kf)
        v20 = jnp.where(hit, m, v20)
        need = jnp.where(hit, kf - cum, need)
        thr = jnp.where(active, m, thr)
        cum = cum + jnp.where(active, cnt, 0.0)
        return cum, thr, v20, need, it + 1

    init = (cnt1, m1, v20_0, need_0, jnp.int32(0))
    _, _, v20, need_f, _ = jax.lax.while_loop(cond, body, init)

    tie = adj == v20
    tie_b = tie.astype(jnp.bfloat16)
    s = jax.lax.dot_general(tie_b, echk_ref[...], dn0,
                            preferred_element_type=jnp.float32)
    p = jax.lax.dot_general(s, tri2_ref[...], dn0,
                            preferred_element_type=jnp.float32)

    ch = s.shape[1]
    cw = epos_ref.shape[1]
    cstar = jnp.sum((p < need_f).astype(jnp.int32), axis=1,
                    keepdims=True) - 1
    onehot = (jax.lax.broadcasted_iota(jnp.int32, (r, ch), 1) == cstar)
    quota = need_f - jnp.sum(jnp.where(onehot, p, 0.0), axis=1, keepdims=True)

    extb = jnp.where(jchunk_ref[...] == cstar, tie_b, jnp.bfloat16(0.0))
    ext = jax.lax.dot_general(extb, epos_ref[...], dn0,
                              preferred_element_type=jnp.float32)
    rank_in = jax.lax.dot_general(ext, tri_ref[...], dn0,
                                  preferred_element_type=jnp.float32)
    hitp = (ext > 0.5) & (rank_in == quota - 1.0)
    pidx = jax.lax.broadcasted_iota(jnp.int32, (r, cw), 1)
    pstar = jnp.sum(jnp.where(hitp, pidx, 0), axis=1, keepdims=True)
    jstar = cstar * cw + pstar

    sel = (adj > v20) | (tie & (jcol_ref[...] <= jstar))
    out_ref[...] = jnp.where(sel, adj, 0.0)


def kernel(idx, emb1, emb2, W1, b1, W2, b2):
    n = idx.shape[0]
    d = emb1.shape[1]
    tab = jnp.concatenate(
        [emb1, emb2, jnp.zeros((n, 128 - 2 * d), jnp.float32)], axis=1)
    gathered = _sc_gather(tab, idx.astype(jnp.int32))
    e1 = gathered[:, :d]
    e2 = gathered[:, d:2 * d]
    b1r = b1.reshape(1, d)
    b2r = b2.reshape(1, d)

    c1, c2 = pl.pallas_call(
        _nodevec_body,
        out_shape=(jax.ShapeDtypeStruct((n, 2 * d), jnp.float32),
                   jax.ShapeDtypeStruct((n, 2 * d), jnp.float32)),
    )(e1, e2, W1, b1r, W2, b2r)

    cw = _pick_cw(n)
    ch = n // cw
    jn = jnp.arange(n, dtype=jnp.int32)
    epos = (jn[:, None] % cw == jnp.arange(cw)[None, :]).astype(jnp.bfloat16)
    echk = (jn[:, None] // cw == jnp.arange(ch)[None, :]).astype(jnp.bfloat16)
    tri = (jnp.arange(cw)[:, None] < jnp.arange(cw)[None, :]).astype(jnp.float32)
    tri2 = (jnp.arange(ch)[:, None] < jnp.arange(ch)[None, :]).astype(jnp.float32)
    jchunk = (jn // cw).astype(jnp.int32).reshape(1, n)
    jcol = jn.reshape(1, n)

    rows = 200 if n % 200 == 0 else 8
    grid = n // rows
    adj = pl.pallas_call(
        _adj_body,
        compiler_params=pltpu.CompilerParams(
            vmem_limit_bytes=100 * 1024 * 1024),
        grid=(grid,),
        in_specs=[
            pl.BlockSpec((rows, 2 * d), lambda i: (i, 0)),
            pl.BlockSpec((n, 2 * d), lambda i: (0, 0)),
            pl.BlockSpec((n, cw), lambda i: (0, 0)),
            pl.BlockSpec((n, ch), lambda i: (0, 0)),
            pl.BlockSpec((cw, cw), lambda i: (0, 0)),
            pl.BlockSpec((ch, ch), lambda i: (0, 0)),
            pl.BlockSpec((1, n), lambda i: (0, 0)),
            pl.BlockSpec((1, n), lambda i: (0, 0)),
        ],
        out_specs=pl.BlockSpec((rows, n), lambda i: (i, 0)),
        out_shape=jax.ShapeDtypeStruct((n, n), jnp.float32),
    )(c1, c2, epos, echk, tri, tri2, jchunk, jcol)
    return adj


# speedup vs baseline: 1.1238x; 1.0015x over previous
import functools

import jax
import jax.numpy as jnp
from jax import lax
from jax.experimental import pallas as pl
from jax.experimental.pallas import tpu as pltpu
from jax.experimental.pallas import tpu_sc as plsc

ALPHA = 3.0
KTOP = 20
NEG = -3.0e38


def _sc_gather(tab, idx):
    b = idx.shape[0]
    d = tab.shape[1]
    try:
        info = plsc.get_sparse_core_info()
        nc, ns = info.num_cores, info.num_subcores
    except ValueError:
        nc, ns = 2, 16
    nw = nc * ns
    chunk = 104 if b >= 104 * nw else 8
    nch = b // (chunk * nw)
    bpw = chunk * nch
    tail = b - bpw * nw
    mesh = plsc.VectorSubcoreMesh(core_axis_name="c", subcore_axis_name="s",
                                  num_cores=nc, num_subcores=ns)

    @functools.partial(
        pl.kernel, mesh=mesh,
        out_type=jax.ShapeDtypeStruct((b, d), jnp.float32),
        scratch_types=[
            pltpu.VMEM((chunk,), jnp.int32),
            pltpu.VMEM((chunk, d), jnp.float32),
            pltpu.VMEM((max(tail, 8),), jnp.int32),
            pltpu.VMEM((max(tail, 8), d), jnp.float32),
            pltpu.SemaphoreType.DMA,
        ],
    )
    def k(tab_hbm, idx_hbm, o_hbm, idx_v, rows_v, idxt_v, rowst_v, sem):
        wid = lax.axis_index("s") * nc + lax.axis_index("c")
        for ci in range(nch):
            base = wid * bpw + ci * chunk
            pltpu.sync_copy(idx_hbm.at[pl.ds(base, chunk)], idx_v)
            pltpu.async_copy(tab_hbm.at[idx_v], rows_v, sem).wait()
            pltpu.sync_copy(rows_v, o_hbm.at[pl.ds(base, chunk)])
        if tail:
            @pl.when(wid == 0)
            def _tail():
                tb = nw * bpw
                pltpu.sync_copy(idx_hbm.at[pl.ds(tb, tail)], idxt_v)
                pltpu.async_copy(tab_hbm.at[idxt_v], rowst_v, sem).wait()
                pltpu.sync_copy(rowst_v, o_hbm.at[pl.ds(tb, tail)])

    return k(tab, idx)


def _nodevec_body(emb1_ref, emb2_ref, w1_ref, b1_ref, w2_ref, b2_ref,
                  c1_ref, c2_ref):
    dn = (((1,), (1,)), ((), ()))
    y1 = jax.lax.dot_general(emb1_ref[...], w1_ref[...], dn,
                             preferred_element_type=jnp.float32)
    y2 = jax.lax.dot_general(emb2_ref[...], w2_ref[...], dn,
                             preferred_element_type=jnp.float32)
    n1 = jnp.tanh(ALPHA * (y1 + b1_ref[...]))
    n2 = jnp.tanh(ALPHA * (y2 + b2_ref[...]))
    c1_ref[...] = jnp.concatenate([n1, -n2], axis=1)
    c2_ref[...] = jnp.concatenate([n2, n1], axis=1)


def _pick_cw(n):
    best = 1
    d = 1
    while d * d <= n:
        if n % d == 0:
            best = d
        d += 1
    return best


def _adj_body(c1b_ref, c2_ref, epos_ref, echk_ref, tri_ref, tri2_ref,
              jchunk_ref, jcol_ref, out_ref):
    r = c1b_ref.shape[0]
    n = c2_ref.shape[0]
    dn1 = (((1,), (1,)), ((), ()))
    dn0 = (((1,), (0,)), ((), ()))
    a = jax.lax.dot_general(c1b_ref[...], c2_ref[...], dn1,
                            preferred_element_type=jnp.float32)
    adj = jnp.where(a > 0.0, jnp.tanh(ALPHA * a), 0.0)

    kf = jnp.float32(KTOP)
    m1 = jnp.max(adj, axis=1, keepdims=True)
    cnt1 = jnp.sum((adj >= m1).astype(jnp.float32), axis=1, keepdims=True)
    hit1 = cnt1 >= kf
    v20_0 = jnp.where(hit1, m1, 0.0)
    need_0 = jnp.where(hit1, kf, 0.0)

    def cond(st):
        cum = st[0]
        it = st[-1]
        return jnp.logical_and(jnp.any(cum < kf), it < KTOP)

    def body(st):
        cum, thr, v20, need, it = st
        m = jnp.max(jnp.where(adj < thr, adj, NEG), axis=1, keepdims=True)
        cnt = jnp.sum((adj == m).astype(jnp.float32), axis=1, keepdims=True)
        active = cum < kf
        hit = active & (cum + cnt >= kf)
        v20 = jnp.where(hit, m, v20)
        need = jnp.where(hit, kf - cum, need)
        thr = jnp.where(active, m, thr)
        cum = cum + jnp.where(active, cnt, 0.0)
        return cum, thr, v20, need, it + 1

    init = (cnt1, m1, v20_0, need_0, jnp.int32(0))
    _, _, v20, need_f, _ = jax.lax.while_loop(cond, body, init)

    tie = adj == v20
    tie_b = tie.astype(jnp.bfloat16)
    s = jax.lax.dot_general(tie_b, echk_ref[...], dn0,
                            preferred_element_type=jnp.float32)
    p = jax.lax.dot_general(s, tri2_ref[...], dn0,
                            preferred_element_type=jnp.float32)

    ch = s.shape[1]
    cw = epos_ref.shape[1]
    cstar = jnp.sum((p < need_f).astype(jnp.int32), axis=1,
                    keepdims=True) - 1
    onehot = (jax.lax.broadcasted_iota(jnp.int32, (r, ch), 1) == cstar)
    quota = need_f - jnp.sum(jnp.where(onehot, p, 0.0), axis=1, keepdims=True)

    extb = jnp.where(jchunk_ref[...] == cstar, tie_b, jnp.bfloat16(0.0))
    ext = jax.lax.dot_general(extb, epos_ref[...], dn0,
                              preferred_element_type=jnp.float32)
    rank_in = jax.lax.dot_general(ext, tri_ref[...], dn0,
                                  preferred_element_type=jnp.float32)
    hitp = (ext > 0.5) & (rank_in == quota - 1.0)
    pidx = jax.lax.broadcasted_iota(jnp.int32, (r, cw), 1)
    pstar = jnp.sum(jnp.where(hitp, pidx, 0), axis=1, keepdims=True)
    jstar = cstar * cw + pstar

    sel = (adj > v20) | (tie & (jcol_ref[...] <= jstar))
    out_ref[...] = jnp.where(sel, adj, 0.0)


def kernel(idx, emb1, emb2, W1, b1, W2, b2):
    n = idx.shape[0]
    d = emb1.shape[1]
    tab = jnp.concatenate(
        [emb1, emb2, jnp.zeros((n, 128 - 2 * d), jnp.float32)], axis=1)
    gathered = _sc_gather(tab, idx.astype(jnp.int32))
    e1 = gathered[:, :d]
    e2 = gathered[:, d:2 * d]
    b1r = b1.reshape(1, d)
    b2r = b2.reshape(1, d)

    c1, c2 = pl.pallas_call(
        _nodevec_body,
        out_shape=(jax.ShapeDtypeStruct((n, 2 * d), jnp.float32),
                   jax.ShapeDtypeStruct((n, 2 * d), jnp.float32)),
    )(e1, e2, W1, b1r, W2, b2r)

    cw = _pick_cw(n)
    ch = n // cw
    jn = jnp.arange(n, dtype=jnp.int32)
    epos = (jn[:, None] % cw == jnp.arange(cw)[None, :]).astype(jnp.bfloat16)
    echk = (jn[:, None] // cw == jnp.arange(ch)[None, :]).astype(jnp.bfloat16)
    tri = (jnp.arange(cw)[:, None] < jnp.arange(cw)[None, :]).astype(jnp.float32)
    tri2 = (jnp.arange(ch)[:, None] < jnp.arange(ch)[None, :]).astype(jnp.float32)
    jchunk = (jn // cw).astype(jnp.int32).reshape(1, n)
    jcol = jn.reshape(1, n)

    rows = 200 if n % 200 == 0 else 8
    grid = n // rows
    adj = pl.pallas_call(
        _adj_body,
        grid=(grid,),
        in_specs=[
            pl.BlockSpec((rows, 2 * d), lambda i: (i, 0)),
            pl.BlockSpec((n, 2 * d), lambda i: (0, 0)),
            pl.BlockSpec((n, cw), lambda i: (0, 0)),
            pl.BlockSpec((n, ch), lambda i: (0, 0)),
            pl.BlockSpec((cw, cw), lambda i: (0, 0)),
            pl.BlockSpec((ch, ch), lambda i: (0, 0)),
            pl.BlockSpec((1, n), lambda i: (0, 0)),
            pl.BlockSpec((1, n), lambda i: (0, 0)),
        ],
        out_specs=pl.BlockSpec((rows, n), lambda i: (i, 0)),
        out_shape=jax.ShapeDtypeStruct((n, n), jnp.float32),
    )(c1, c2, epos, echk, tri, tri2, jchunk, jcol)
    return adj
